# Initial kernel scaffold; baseline (speedup 1.0000x reference)
import jax
import jax.numpy as jnp
from jax.experimental import pallas as pl


def kernel(x, edge_index, W1, b1, W2, b2):
    raise NotImplementedError("write your pallas kernel here")


# speedup vs baseline: 18.5943x; 18.5943x over previous
import functools

import jax
import jax.numpy as jnp
from jax import lax
from jax.experimental import pallas as pl
from jax.experimental.pallas import tpu as pltpu
from jax.experimental.pallas import tpu_sc as plsc

N = 10000
E = 320000
NPAD = 10240
NW = 32
EPW = E // NW
CH = 128
NCH = EPW // CH
TAIL = EPW - NCH * CH
RPT = NPAD // 16


def _worker_id():
    cid = lax.axis_index("c")
    sid = lax.axis_index("s")
    return cid, sid, sid * 2 + cid


def _zero_acc(zbuf, acc, sid, width):
    def zrow(r, carry):
        for c in range(width // 16):
            zbuf[r, pl.ds(16 * c, 16)] = jnp.zeros((16,), jnp.float32)
        return carry

    lax.fori_loop(0, CH, zrow, 0)
    for k in range(RPT // CH):
        pltpu.sync_copy(zbuf, acc.at[pl.ds(sid * RPT + k * CH, CH)])


def _make_deg_kernel():
    mesh = plsc.VectorSubcoreMesh(core_axis_name="c", subcore_axis_name="s")

    @functools.partial(
        pl.kernel,
        out_type=jax.ShapeDtypeStruct((2, NPAD, 16), jnp.float32),
        mesh=mesh,
        scratch_types=[
            pltpu.VMEM((CH,), jnp.int32),
            pltpu.VMEM((TAIL,), jnp.int32),
            pltpu.VMEM((CH, 16), jnp.float32),
            pltpu.VMEM((CH, 16), jnp.float32),
            pltpu.VMEM_SHARED((NPAD, 16), jnp.float32),
        ],
    )
    def k(dst_hbm, out_hbm, didx, didx_t, ones_v, zbuf, acc):
        cid, sid, wid = _worker_id()

        def frow(r, carry):
            ones_v[r, :] = jnp.ones((16,), jnp.float32)
            return carry

        lax.fori_loop(0, CH, frow, 0)
        _zero_acc(zbuf, acc, sid, 16)
        plsc.subcore_barrier()

        base = wid * EPW

        def body(j, carry):
            pltpu.sync_copy(dst_hbm.at[pl.ds(base + j * CH, CH)], didx)
            pltpu.sync_copy(ones_v, acc.at[didx], add=True)
            return carry

        lax.fori_loop(0, NCH, body, 0)
        pltpu.sync_copy(dst_hbm.at[pl.ds(base + NCH * CH, TAIL)], didx_t)
        pltpu.sync_copy(ones_v.at[pl.ds(0, TAIL)], acc.at[didx_t], add=True)

        plsc.subcore_barrier()
        pltpu.sync_copy(acc.at[pl.ds(sid * RPT, RPT)],
                        out_hbm.at[cid, pl.ds(sid * RPT, RPT)])

    return k


def _make_agg_kernel(width):
    mesh = plsc.VectorSubcoreMesh(core_axis_name="c", subcore_axis_name="s")

    @functools.partial(
        pl.kernel,
        out_type=jax.ShapeDtypeStruct((2, NPAD, width), jnp.float32),
        mesh=mesh,
        compiler_params=pltpu.CompilerParams(
            use_tc_tiling_on_sc=(width % 128 == 0)),
        scratch_types=[
            pltpu.VMEM((CH,), jnp.int32),
            pltpu.VMEM((CH,), jnp.int32),
            pltpu.VMEM((TAIL,), jnp.int32),
            pltpu.VMEM((TAIL,), jnp.int32),
            pltpu.VMEM((CH, width), jnp.float32),
            pltpu.VMEM((TAIL, width), jnp.float32),
            pltpu.VMEM((CH, width), jnp.float32),
            pltpu.VMEM_SHARED((NPAD, width), jnp.float32),
            pltpu.SemaphoreType.DMA,
        ],
    )
    def k(table_hbm, src_hbm, dst_hbm, out_hbm,
          sidx, didx, sidx_t, didx_t, rows, rows_t, zbuf, acc, sem):
        cid, sid, wid = _worker_id()
        _zero_acc(zbuf, acc, sid, width)
        plsc.subcore_barrier()

        base = wid * EPW

        def body(j, carry):
            off = base + j * CH
            pltpu.sync_copy(src_hbm.at[pl.ds(off, CH)], sidx)
            pltpu.sync_copy(dst_hbm.at[pl.ds(off, CH)], didx)
            pltpu.async_copy(table_hbm.at[sidx], rows, sem).wait()
            pltpu.sync_copy(rows, acc.at[didx], add=True)
            return carry

        lax.fori_loop(0, NCH, body, 0)
        off = base + NCH * CH
        pltpu.sync_copy(src_hbm.at[pl.ds(off, TAIL)], sidx_t)
        pltpu.sync_copy(dst_hbm.at[pl.ds(off, TAIL)], didx_t)
        pltpu.async_copy(table_hbm.at[sidx_t], rows_t, sem).wait()
        pltpu.sync_copy(rows_t, acc.at[didx_t], add=True)

        plsc.subcore_barrier()
        pltpu.sync_copy(acc.at[pl.ds(sid * RPT, RPT)],
                        out_hbm.at[cid, pl.ds(sid * RPT, RPT)])

    return k


def _tc_scale(degp_ref, x_ref, xp_ref, disv_ref):
    dval = degp_ref[0, :N, 0:1] + degp_ref[1, :N, 0:1] + 1.0
    dis = lax.rsqrt(dval)
    xp_ref[...] = x_ref[...] * dis
    disv_ref[...] = jnp.broadcast_to(dis, (N, 16))


def _tc_mid(agg1_ref, xp_ref, disv_ref, w1_ref, b1_ref, w2p_ref, p_ref):
    a = agg1_ref[0, :N, :] + agg1_ref[1, :N, :] + xp_ref[...]
    dis = disv_ref[:, 0:1]
    z = jnp.dot(a, w1_ref[...], preferred_element_type=jnp.float32) * dis + b1_ref[...]
    h = jnp.where(z >= 0.0, z, 0.2 * z)
    p_ref[...] = jnp.dot(h * dis, w2p_ref[...], preferred_element_type=jnp.float32)


def _tc_final(agg2_ref, p_ref, disv_ref, b2p_ref, out_ref):
    s = agg2_ref[0, :N, :] + agg2_ref[1, :N, :] + p_ref[...]
    z = s * disv_ref[:, 0:1] + b2p_ref[...]
    z = z - jnp.max(z, axis=1, keepdims=True)
    out_ref[...] = z - jnp.log(jnp.sum(jnp.exp(z), axis=1, keepdims=True))


def kernel(x, edge_index, W1, b1, W2, b2):
    src = edge_index[0].astype(jnp.int32)
    dst = edge_index[1].astype(jnp.int32)
    w2p = jnp.pad(W2, ((0, 0), (0, 16 - W2.shape[1])))
    b2p = jnp.concatenate([b2, jnp.full((16 - b2.shape[0],), -1e30, b2.dtype)])

    degp = _make_deg_kernel()(dst)
    xp, disv = pl.pallas_call(
        _tc_scale,
        out_shape=[jax.ShapeDtypeStruct((N, 128), jnp.float32),
                   jax.ShapeDtypeStruct((N, 16), jnp.float32)],
    )(degp, x)
    agg1 = _make_agg_kernel(128)(xp, src, dst)
    p = pl.pallas_call(
        _tc_mid,
        out_shape=jax.ShapeDtypeStruct((N, 16), jnp.float32),
    )(agg1, xp, disv, W1, b1, w2p)
    agg2 = _make_agg_kernel(16)(p, src, dst)
    out16 = pl.pallas_call(
        _tc_final,
        out_shape=jax.ShapeDtypeStruct((N, 16), jnp.float32),
    )(agg2, p, disv, b2p)
    return out16[:, :10]
